# baseline (speedup 1.0000x reference)
import functools

import jax
import jax.numpy as jnp
from jax import lax
from jax.experimental import pallas as pl
from jax.experimental.pallas import tpu as pltpu
from jax.experimental.pallas import tpu_sc as plsc

NC = 2
NS = 16
NW = NC * NS

CHUNK = 128
RING_D = 8
RING_LAG = 2
IB = 16


def _mesh():
  return plsc.VectorSubcoreMesh(
      core_axis_name="c", subcore_axis_name="s", num_cores=NC,
      num_subcores=NS)


def _fill_zeros(ref, n_vec):
  zero = jnp.zeros((16,), jnp.float32)

  def body(i, _):
    ref[pl.ds(i * 16, 16)] = zero
    return 0

  lax.fori_loop(0, n_vec, body, 0)


def _sc_agg1_body(x_hbm, src_hbm, dst_hbm, out_hbm,
                  acc_sh, src_v, dst_v, vals_v, zbuf, gsem, ssem, isem):
  c = lax.axis_index("c")
  s = lax.axis_index("s")
  wid = c * NS + s

  n_acc = out_hbm.shape[1]
  rows_tile = n_acc // NS

  _fill_zeros(zbuf, rows_tile // 16)
  pltpu.sync_copy(zbuf, acc_sh.at[pl.ds(s * rows_tile, rows_tile)])
  plsc.subcore_barrier()

  n_rows = src_hbm.shape[0] // NW
  row_base = wid * n_rows
  _pipeline(lambda idx: x_hbm.at[idx], src_hbm, dst_hbm, acc_sh,
            src_v, dst_v, vals_v, gsem, ssem, isem, row_base, n_rows)
  plsc.subcore_barrier()

  pltpu.sync_copy(acc_sh.at[pl.ds(s * rows_tile, rows_tile)],
                  out_hbm.at[c].at[pl.ds(s * rows_tile, rows_tile)])


def _sc_agg2_body(h1_hbm, src_hbm, dst_hbm, out_hbm,
                  acc_sh, src_v, dst_v, vals_v, zbuf, gsem, ssem, isem):
  c = lax.axis_index("c")
  s = lax.axis_index("s")

  n_acc = out_hbm.shape[1]
  rows_tile = n_acc // NS

  zrows = zbuf.shape[0]
  zero = jnp.zeros((16,), jnp.float32)

  def zbody(i, _):
    zbuf[i, :] = zero
    return 0

  lax.fori_loop(0, zrows, zbody, 0)
  for k in range(rows_tile // zrows):
    pltpu.sync_copy(zbuf, acc_sh.at[pl.ds(s * rows_tile + k * zrows, zrows)])
  plsc.subcore_barrier()

  n_rows = src_hbm.shape[0] // NS
  row_base = s * n_rows
  _pipeline(lambda idx: h1_hbm.at[c].at[idx], src_hbm, dst_hbm, acc_sh,
            src_v, dst_v, vals_v, gsem, ssem, isem, row_base, n_rows)
  plsc.subcore_barrier()

  pltpu.sync_copy(acc_sh.at[pl.ds(s * rows_tile, rows_tile)],
                  out_hbm.at[c].at[pl.ds(s * rows_tile, rows_tile)])


def _pipeline(gsrc, src_hbm, dst_hbm, acc_sh, src_v, dst_v, vals_v,
              gsem, ssem, isem, row_base, n_rows):
  ib = src_v.shape[1]
  d = vals_v.shape[0]
  g = d - RING_LAG
  lagn = d - g
  n_blocks = n_rows // ib

  def idx_row(ref, r):
    return ref.at[(r // ib) % 2].at[r % ib]

  def fire_idx(blk):
    buf = blk % 2
    r0 = row_base + blk * ib
    pltpu.async_copy(src_hbm.at[pl.ds(r0, ib)], src_v.at[buf], isem)
    pltpu.async_copy(dst_hbm.at[pl.ds(r0, ib)], dst_v.at[buf], isem)

  def wait_idx(blk):
    buf = blk % 2
    r0 = row_base + blk * ib
    pltpu.make_async_copy(src_hbm.at[pl.ds(r0, ib)], src_v.at[buf],
                          isem).wait()
    pltpu.make_async_copy(dst_hbm.at[pl.ds(r0, ib)], dst_v.at[buf],
                          isem).wait()

  def fire_g(r):
    pltpu.async_copy(gsrc(idx_row(src_v, r)), vals_v.at[r % d], gsem)

  def wait_g(r):
    pltpu.make_async_copy(gsrc(idx_row(src_v, r)), vals_v.at[r % d],
                          gsem).wait()

  def fire_s(r):
    pltpu.async_copy(vals_v.at[r % d], acc_sh.at[idx_row(dst_v, r)],
                     ssem, add=True)

  def wait_s(r):
    pltpu.make_async_copy(vals_v.at[r % d], acc_sh.at[idx_row(dst_v, r)],
                          ssem).wait()

  fire_idx(0)
  wait_idx(0)
  if n_blocks > 1:
    fire_idx(1)
  for r in range(g):
    fire_g(r)

  def body(r, _):
    lag = r - lagn

    @pl.when(lag >= 0)
    def _():
      wait_s(lag)

    wait_g(r)
    fire_s(r)

    @pl.when((lax.rem(r, ib) == lagn) & (r // ib + 2 < n_blocks + 1)
             & (r // ib >= 1))
    def _():
      fire_idx(r // ib + 2 - 1)

    rn = r + g

    @pl.when(rn < n_rows)
    def _():
      @pl.when(lax.rem(rn, ib) == 0)
      def _():
        wait_idx(rn // ib)

      fire_g(rn)

    return 0

  lax.fori_loop(0, n_rows, body, 0)
  for j in range(lagn):
    wait_s(n_rows - lagn + j)


def _make_sc_agg1(n, n_acc, e_pad):
  return pl.kernel(
      _sc_agg1_body,
      out_type=jax.ShapeDtypeStruct((NC, n_acc), jnp.float32),
      mesh=_mesh(),
      compiler_params=pltpu.CompilerParams(use_tc_tiling_on_sc=False),
      scratch_types=[
          pltpu.VMEM_SHARED((n_acc,), jnp.float32),
          pltpu.VMEM((2, IB, CHUNK), jnp.int32),
          pltpu.VMEM((2, IB, CHUNK), jnp.int32),
          pltpu.VMEM((RING_D, CHUNK), jnp.float32),
          pltpu.VMEM((n_acc // NS,), jnp.float32),
          pltpu.SemaphoreType.DMA,
          pltpu.SemaphoreType.DMA,
          pltpu.SemaphoreType.DMA,
      ],
  )


def _make_sc_agg2(n, n_acc, e_pad, hh):
  zrows = (n_acc // NS) // 64
  return pl.kernel(
      _sc_agg2_body,
      out_type=jax.ShapeDtypeStruct((NC, n_acc, hh), jnp.float32),
      mesh=_mesh(),
      compiler_params=pltpu.CompilerParams(use_tc_tiling_on_sc=False),
      scratch_types=[
          pltpu.VMEM_SHARED((n_acc, hh), jnp.float32),
          pltpu.VMEM((2, IB, CHUNK), jnp.int32),
          pltpu.VMEM((2, IB, CHUNK), jnp.int32),
          pltpu.VMEM((RING_D, CHUNK, hh), jnp.float32),
          pltpu.VMEM((zrows, hh), jnp.float32),
          pltpu.SemaphoreType.DMA,
          pltpu.SemaphoreType.DMA,
          pltpu.SemaphoreType.DMA,
      ],
  )


def _tc_mlp1_body(x_ref, agg_ref, w1a_ref, b1a_ref, w1b_ref, b1b_ref,
                  out_ref):
  t = x_ref[:, 0] + agg_ref[0, :, 0] + agg_ref[1, :, 0]
  h = jnp.maximum(t[:, None] * w1a_ref[0, :][None, :] + b1a_ref[0, :][None, :],
                  0.0)
  h = jnp.dot(h, w1b_ref[:, :], preferred_element_type=jnp.float32)
  h = jnp.maximum(h + b1b_ref[0, :][None, :], 0.0)
  hh = out_ref.shape[2]
  out_ref[0] = h[:, :hh]
  out_ref[1] = h[:, hh:]


def _tc_mlp2_body(h1_ref, agg_ref, w2a_ref, b2a_ref, w2b_ref, b2b_ref,
                  out_ref):
  hh = jnp.concatenate(
      [h1_ref[0] + agg_ref[0], h1_ref[1] + agg_ref[1]], axis=1)
  z = jnp.dot(hh, w2a_ref[:, :], preferred_element_type=jnp.float32)
  z = jnp.maximum(z + b2a_ref[0, :][None, :], 0.0)
  z = jnp.dot(z, w2b_ref[:, :], preferred_element_type=jnp.float32)
  z = jnp.maximum(z + b2b_ref[0, :][None, :], 0.0)
  out_ref[0, 0, :] = jnp.sum(z, axis=0)


def _run_graph(x, edge_index, params, n, h, n_acc, e_pad, blk):
  (w1a, b1a, w1b, b1b, w2a, b2a, w2b, b2b) = params
  hh = h // 2
  e = edge_index.shape[1]

  pad = e_pad - e
  src = jnp.concatenate([edge_index[0], jnp.zeros((pad,), jnp.int32)])
  dst = jnp.concatenate(
      [edge_index[1], jnp.full((pad,), n, jnp.int32)])
  src2 = src.reshape(e_pad // CHUNK, CHUNK)
  dst2 = dst.reshape(e_pad // CHUNK, CHUNK)
  xf = x.reshape(n)

  agg1 = _make_sc_agg1(n, n_acc, e_pad)(xf, src2, dst2)

  grid = n // blk
  h1s = pl.pallas_call(
      _tc_mlp1_body,
      grid=(grid,),
      in_specs=[
          pl.BlockSpec((blk, 1), lambda i: (i, 0)),
          pl.BlockSpec((NC, blk, 1), lambda i: (0, i, 0)),
          pl.BlockSpec((1, h), lambda i: (0, 0)),
          pl.BlockSpec((1, h), lambda i: (0, 0)),
          pl.BlockSpec((h, h), lambda i: (0, 0)),
          pl.BlockSpec((1, h), lambda i: (0, 0)),
      ],
      out_specs=pl.BlockSpec((NC, blk, hh), lambda i: (0, i, 0)),
      out_shape=jax.ShapeDtypeStruct((NC, n, hh), jnp.float32),
  )(x, agg1.reshape(NC, n_acc, 1), w1a, b1a.reshape(1, h), w1b,
    b1b.reshape(1, h))

  agg2 = _make_sc_agg2(n, n_acc, e_pad, hh)(h1s, src2, dst2)

  psums = pl.pallas_call(
      _tc_mlp2_body,
      grid=(grid,),
      in_specs=[
          pl.BlockSpec((NC, blk, hh), lambda i: (0, i, 0)),
          pl.BlockSpec((NC, blk, hh), lambda i: (0, i, 0)),
          pl.BlockSpec((h, h), lambda i: (0, 0)),
          pl.BlockSpec((1, h), lambda i: (0, 0)),
          pl.BlockSpec((h, h), lambda i: (0, 0)),
          pl.BlockSpec((1, h), lambda i: (0, 0)),
      ],
      out_specs=pl.BlockSpec((1, 1, h), lambda i: (i, 0, 0)),
      out_shape=jax.ShapeDtypeStruct((grid, 1, h), jnp.float32),
  )(h1s, agg2, w2a, b2a.reshape(1, h), w2b, b2b.reshape(1, h))

  return jnp.sum(psums.reshape(grid, h), axis=0)


@jax.jit
def _kernel_impl(x_anchor, edge_index_anchor, x_positive,
                 edge_index_positive, x_negative, edge_index_negative,
                 W1a, b1a, W1b, b1b, W2a, b2a, W2b, b2b, Wf, bf):
  n = x_anchor.shape[0]
  h = W1b.shape[0]
  e = edge_index_anchor.shape[1]

  unit = NS * 16 * 8
  n_acc = ((n + 16) + unit - 1) // unit * unit

  unit_e = NW * IB * CHUNK
  e_pad = (e + unit_e - 1) // unit_e * unit_e

  blk = 1000
  assert n % blk == 0

  params = (W1a, b1a, W1b, b1b, W2a, b2a, W2b, b2b)
  outs = []
  for x, ei in ((x_anchor, edge_index_anchor),
                (x_positive, edge_index_positive),
                (x_negative, edge_index_negative)):
    s = _run_graph(x, ei, params, n, h, n_acc, e_pad, blk)
    outs.append(s @ Wf + bf)
  return tuple(outs)


def kernel(x_anchor, edge_index_anchor, x_positive, edge_index_positive,
           x_negative, edge_index_negative, W1a, b1a, W1b, b1b, W2a, b2a,
           W2b, b2b, Wf, bf):
  return _kernel_impl(
      x_anchor, edge_index_anchor, x_positive, edge_index_positive,
      x_negative, edge_index_negative, W1a, b1a, W1b, b1b, W2a, b2a,
      W2b, b2b, Wf, bf)
